# baseline (speedup 1.0000x reference)
import functools

import jax
import jax.numpy as jnp
from jax import lax
from jax.experimental import pallas as pl
from jax.experimental.pallas import tpu as pltpu
from jax.experimental.pallas import tpu_sc as plsc

HIDDEN = 128
CHUNK = 128
LANES = 16
N_WORKERS = 32


def _sc_embed_sum(ids_flat, x_table, y_table, t_table, n_tokens):
    per_worker = n_tokens // N_WORKERS
    n_chunks = per_worker // CHUNK
    mesh = plsc.VectorSubcoreMesh(core_axis_name="c", subcore_axis_name="s")

    @functools.partial(
        pl.kernel,
        mesh=mesh,
        out_type=jax.ShapeDtypeStruct((n_tokens, HIDDEN), jnp.float32),
        scratch_types=[
            pltpu.VMEM((3 * per_worker,), jnp.int32),
            pltpu.VMEM((per_worker,), jnp.int32),
            pltpu.VMEM((per_worker,), jnp.int32),
            pltpu.VMEM((per_worker,), jnp.int32),
            pltpu.VMEM((CHUNK, HIDDEN), jnp.float32),
            pltpu.VMEM((CHUNK, HIDDEN), jnp.float32),
            pltpu.VMEM((CHUNK, HIDDEN), jnp.float32),
            pltpu.VMEM((CHUNK, HIDDEN), jnp.float32),
            pltpu.VMEM((CHUNK, HIDDEN), jnp.float32),
            pltpu.VMEM((CHUNK, HIDDEN), jnp.float32),
            pltpu.SemaphoreType.DMA,
            pltpu.SemaphoreType.DMA,
            pltpu.SemaphoreType.DMA,
            pltpu.SemaphoreType.DMA,
        ],
    )
    def body(ids_hbm, x_hbm, y_hbm, t_hbm, out_hbm,
             rawv, ixv, iyv, itv, bx0, by0, bt0, bx1, by1, bt1,
             sg0, sg1, ss0, ss1):
        wid = lax.axis_index("s") * 2 + lax.axis_index("c")
        w_base = wid * per_worker
        pltpu.sync_copy(ids_hbm.at[pl.ds(w_base * 3, 3 * per_worker)], rawv)

        lane3 = lax.iota(jnp.int32, LANES) * 3
        dn = lax.GatherDimensionNumbers(
            offset_dims=(), collapsed_slice_dims=(0,), start_index_map=(0,))

        def vgather(v, idx):
            return lax.gather(v, idx[:, None], dn, (1,),
                              mode=lax.GatherScatterMode.PROMISE_IN_BOUNDS)

        sel = []
        for t in range(3):
            p = lane3 + t
            lane = jnp.bitwise_and(p, LANES - 1)
            src = jnp.right_shift(p, 4)
            sel.append((lane, src == 0, src == 1))

        def deint_step(g, _):
            dst = pl.ds(g * LANES, LANES)
            b0 = g * 3 * LANES
            v0 = rawv[pl.ds(b0, LANES)]
            v1 = rawv[pl.ds(b0 + LANES, LANES)]
            v2 = rawv[pl.ds(b0 + 2 * LANES, LANES)]
            for t, ref in ((0, ixv), (1, iyv), (2, itv)):
                lane, m0, m1 = sel[t]
                ref[dst] = jnp.where(
                    m0, vgather(v0, lane),
                    jnp.where(m1, vgather(v1, lane), vgather(v2, lane)))
            return 0

        lax.fori_loop(0, per_worker // LANES, deint_step, 0)

        bufs = ((bx0, by0, bt0, sg0, ss0), (bx1, by1, bt1, sg1, ss1))

        def fire(p, k):
            bx, by, bt, sg, _ = bufs[p]
            sl = pl.ds(k * CHUNK, CHUNK)
            return (pltpu.async_copy(x_hbm.at[ixv.at[sl]], bx, sg),
                    pltpu.async_copy(y_hbm.at[iyv.at[sl]], by, sg),
                    pltpu.async_copy(t_hbm.at[itv.at[sl]], bt, sg))

        gd = [None, None]
        sd = [None, None]
        gd[0] = fire(0, 0)
        for k in range(n_chunks):
            p = k & 1
            q = (k + 1) & 1
            if k + 1 < n_chunks:
                if sd[q] is not None:
                    sd[q].wait()
                gd[q] = fire(q, k + 1)
            for d in gd[p]:
                d.wait()
            bx, by, bt, _, ss = bufs[p]

            def row_step(r, _, bx=bx, by=by, bt=bt):
                for c in range(HIDDEN // LANES):
                    sl = pl.ds(c * LANES, LANES)
                    bx[r, sl] = bx[r, sl] + by[r, sl] + bt[r, sl]
                return 0

            lax.fori_loop(0, CHUNK, row_step, 0)
            sd[p] = pltpu.async_copy(
                bx, out_hbm.at[pl.ds(w_base + k * CHUNK, CHUNK)], ss)
        for d in sd:
            if d is not None:
                d.wait()

    return body(ids_flat, x_table, y_table, t_table)


def kernel(position_ids, x_table, y_table, time_table):
    b, s, _ = position_ids.shape
    n_tokens = b * s
    ids_flat = position_ids.astype(jnp.int32).reshape(n_tokens * 3)
    out = _sc_embed_sum(ids_flat, x_table, y_table, time_table, n_tokens)
    return out.reshape(b, s, HIDDEN)


# speedup vs baseline: 1.4077x; 1.4077x over previous
import functools

import jax
import jax.numpy as jnp
from jax import lax
from jax.experimental import pallas as pl
from jax.experimental.pallas import tpu as pltpu
from jax.experimental.pallas import tpu_sc as plsc

HIDDEN = 128
CHUNK = 128
LANES = 16
N_WORKERS = 32


def _sc_embed_sum(ix, iy, it, x_table, y_table, t_table, n_tokens):
    per_worker = n_tokens // N_WORKERS
    n_chunks = per_worker // CHUNK
    mesh = plsc.VectorSubcoreMesh(core_axis_name="c", subcore_axis_name="s")

    @functools.partial(
        pl.kernel,
        mesh=mesh,
        out_type=jax.ShapeDtypeStruct((n_tokens, HIDDEN), jnp.float32),
        scratch_types=[
            pltpu.VMEM((n_chunks, CHUNK), jnp.int32),
            pltpu.VMEM((n_chunks, CHUNK), jnp.int32),
            pltpu.VMEM((n_chunks, CHUNK), jnp.int32),
            pltpu.VMEM((CHUNK, HIDDEN), jnp.float32),
            pltpu.VMEM((CHUNK, HIDDEN), jnp.float32),
            pltpu.VMEM((CHUNK, HIDDEN), jnp.float32),
            pltpu.VMEM((CHUNK, HIDDEN), jnp.float32),
            pltpu.VMEM((CHUNK, HIDDEN), jnp.float32),
            pltpu.VMEM((CHUNK, HIDDEN), jnp.float32),
            pltpu.SemaphoreType.DMA,
            pltpu.SemaphoreType.DMA,
            pltpu.SemaphoreType.DMA,
            pltpu.SemaphoreType.DMA,
        ],
    )
    def body(ix_hbm, iy_hbm, it_hbm, x_hbm, y_hbm, t_hbm, out_hbm,
             ixv, iyv, itv, bx0, by0, bt0, bx1, by1, bt1,
             sg0, sg1, ss0, ss1):
        wid = lax.axis_index("s") * 2 + lax.axis_index("c")
        w_base = wid * per_worker
        pltpu.sync_copy(ix_hbm.at[wid], ixv)
        pltpu.sync_copy(iy_hbm.at[wid], iyv)
        pltpu.sync_copy(it_hbm.at[wid], itv)

        bufs = ((bx0, by0, bt0, sg0, ss0), (bx1, by1, bt1, sg1, ss1))

        def fire(p, k):
            bx, by, bt, sg, _ = bufs[p]
            return (pltpu.async_copy(x_hbm.at[ixv.at[k]], bx, sg),
                    pltpu.async_copy(y_hbm.at[iyv.at[k]], by, sg),
                    pltpu.async_copy(t_hbm.at[itv.at[k]], bt, sg))

        gd = [None, None]
        sd = [None, None]
        gd[0] = fire(0, 0)
        for k in range(n_chunks):
            p = k & 1
            q = (k + 1) & 1
            if k + 1 < n_chunks:
                if sd[q] is not None:
                    sd[q].wait()
                gd[q] = fire(q, k + 1)
            for d in gd[p]:
                d.wait()
            bx, by, bt, _, ss = bufs[p]

            def row_step(r, _, bx=bx, by=by, bt=bt):
                for c in range(HIDDEN // LANES):
                    sl = pl.ds(c * LANES, LANES)
                    plsc.addupdate(bx.at[r, sl], by[r, sl] + bt[r, sl])
                return 0

            lax.fori_loop(0, CHUNK, row_step, 0)
            sd[p] = pltpu.async_copy(
                bx, out_hbm.at[pl.ds(w_base + k * CHUNK, CHUNK)], ss)
        for d in sd:
            if d is not None:
                d.wait()

    return body(ix, iy, it, x_table, y_table, t_table)


def kernel(position_ids, x_table, y_table, time_table):
    b, s, _ = position_ids.shape
    n_tokens = b * s
    per_worker = n_tokens // N_WORKERS
    n_chunks = per_worker // CHUNK
    ids = position_ids.reshape(n_tokens, 3).astype(jnp.int32)
    ix = ids[:, 0].reshape(N_WORKERS, n_chunks, CHUNK)
    iy = ids[:, 1].reshape(N_WORKERS, n_chunks, CHUNK)
    it = ids[:, 2].reshape(N_WORKERS, n_chunks, CHUNK)
    out = _sc_embed_sum(ix, iy, it, x_table, y_table, time_table, n_tokens)
    return out.reshape(b, s, HIDDEN)


# speedup vs baseline: 1.4298x; 1.0157x over previous
import functools

import jax
import jax.numpy as jnp
from jax import lax
from jax.experimental import pallas as pl
from jax.experimental.pallas import tpu as pltpu
from jax.experimental.pallas import tpu_sc as plsc

HIDDEN = 128
CHUNK = 128
LANES = 16
N_WORKERS = 32


def _sc_embed_sum(ix, x_table, y_table, t_table, n_tokens):
    per_worker = n_tokens // N_WORKERS
    n_chunks = per_worker // CHUNK
    mesh = plsc.VectorSubcoreMesh(core_axis_name="c", subcore_axis_name="s")

    @functools.partial(
        pl.kernel,
        mesh=mesh,
        out_type=jax.ShapeDtypeStruct((n_tokens, HIDDEN), jnp.float32),
        scratch_types=[
            pltpu.VMEM((3, n_chunks, CHUNK), jnp.int32),
            pltpu.VMEM((CHUNK, HIDDEN), jnp.float32),
            pltpu.VMEM((CHUNK, HIDDEN), jnp.float32),
            pltpu.VMEM((CHUNK, HIDDEN), jnp.float32),
            pltpu.VMEM((CHUNK, HIDDEN), jnp.float32),
            pltpu.VMEM((CHUNK, HIDDEN), jnp.float32),
            pltpu.VMEM((CHUNK, HIDDEN), jnp.float32),
            pltpu.SemaphoreType.DMA,
            pltpu.SemaphoreType.DMA,
            pltpu.SemaphoreType.DMA,
            pltpu.SemaphoreType.DMA,
        ],
    )
    def body(idx_hbm, x_hbm, y_hbm, t_hbm, out_hbm,
             iv, bx0, by0, bt0, bx1, by1, bt1,
             sg0, sg1, ss0, ss1):
        wid = lax.axis_index("s") * 2 + lax.axis_index("c")
        w_base = wid * per_worker
        pltpu.sync_copy(idx_hbm.at[wid], iv)

        bufs = ((bx0, by0, bt0, sg0, ss0), (bx1, by1, bt1, sg1, ss1))

        def fire(p, k):
            bx, by, bt, sg, _ = bufs[p]
            return (pltpu.async_copy(x_hbm.at[iv.at[0, k]], bx, sg),
                    pltpu.async_copy(y_hbm.at[iv.at[1, k]], by, sg),
                    pltpu.async_copy(t_hbm.at[iv.at[2, k]], bt, sg))

        gd = [None, None]
        sd = [None, None]
        gd[0] = fire(0, 0)
        for k in range(n_chunks):
            p = k & 1
            q = (k + 1) & 1
            if k + 1 < n_chunks:
                if sd[q] is not None:
                    sd[q].wait()
                gd[q] = fire(q, k + 1)
            for d in gd[p]:
                d.wait()
            bx, by, bt, _, ss = bufs[p]

            def row_step(r, _, bx=bx, by=by, bt=bt):
                for c in range(HIDDEN // LANES):
                    sl = pl.ds(c * LANES, LANES)
                    plsc.addupdate(bx.at[r, sl], by[r, sl] + bt[r, sl])
                return 0

            lax.fori_loop(0, CHUNK, row_step, 0)
            sd[p] = pltpu.async_copy(
                bx, out_hbm.at[pl.ds(w_base + k * CHUNK, CHUNK)], ss)
        for d in sd:
            if d is not None:
                d.wait()

    return body(ix, x_table, y_table, t_table)


def kernel(position_ids, x_table, y_table, time_table):
    b, s, _ = position_ids.shape
    n_tokens = b * s
    per_worker = n_tokens // N_WORKERS
    n_chunks = per_worker // CHUNK
    ids = position_ids.reshape(n_tokens, 3).astype(jnp.int32)
    idx = (ids.T.reshape(3, N_WORKERS, n_chunks, CHUNK)
           .transpose(1, 0, 2, 3))
    out = _sc_embed_sum(idx, x_table, y_table, time_table, n_tokens)
    return out.reshape(b, s, HIDDEN)
